# baseline (speedup 1.0000x reference)
import functools

import jax
import jax.numpy as jnp
from jax import lax
from jax.experimental import pallas as pl
from jax.experimental.pallas import tpu as pltpu
from jax.experimental.pallas import tpu_sc as plsc

E = 320000
D = 128
N = 10000

_info = plsc.get_sparse_core_info()
NC, NS, L = _info.num_cores, _info.num_subcores, _info.num_lanes
NT = 8
NEG = 4
P = D // 2 // NT
E_GRP = E // NEG
C_E = 4000
N_CH = E_GRP // C_E
G = C_E // L
R = C_E // L
_MASKHI = -65536
_NROW = 512
_NPAD = 20 * _NROW


def _pack_body(h_ref, o_ref):
    x = h_ref[...]
    u16 = lax.bitcast_convert_type(x.astype(jnp.bfloat16), jnp.uint16)
    u = u16.astype(jnp.int32).reshape(_NROW, NT * P, 2)
    w = (u[:, :, 0] << 16) | u[:, :, 1]
    o_ref[...] = w.T


_pack = pl.pallas_call(
    _pack_body,
    grid=(_NPAD // _NROW,),
    in_specs=[pl.BlockSpec((_NROW, D), lambda i: (i, 0))],
    out_specs=pl.BlockSpec((NT * P, _NROW), lambda i: (0, i)),
    out_shape=jax.ShapeDtypeStruct((NT * P, _NPAD), jnp.int32),
)


def _make_sc_kernel():
    mesh = plsc.VectorSubcoreMesh(core_axis_name="c", subcore_axis_name="s")

    @functools.partial(
        pl.kernel,
        mesh=mesh,
        out_type=jax.ShapeDtypeStruct((E // L, L), jnp.float32),
        compiler_params=pltpu.CompilerParams(
            needs_layout_passes=False, use_tc_tiling_on_sc=False
        ),
        scratch_types=[
            pltpu.VMEM((P, _NPAD), jnp.int32),
            pltpu.VMEM((2, C_E), jnp.int32),
            pltpu.VMEM((2, C_E), jnp.int32),
            pltpu.VMEM((2, R, L), jnp.float32),
            pltpu.VMEM((4, R), jnp.int32),
            pltpu.VMEM_SHARED((4 * R, L), jnp.float32),
            pltpu.SemaphoreType.DMA((2,)),
            pltpu.SemaphoreType.DMA((2,)),
        ],
    )
    def edge_score(hp_hbm, src_hbm, dst_hbm, rows_hbm, out_hbm,
                   table_v, src_idx, dst_idx, partial_v, rows_v, acc_sh,
                   sem_idx, sem_out):
        core = lax.axis_index("c")
        s = lax.axis_index("s")
        t = lax.rem(s, NT)
        leg = s // NT
        eg = core * 2 + leg
        slot = leg * 2
        base = eg * E_GRP

        def start_idx(c, b):
            off = base + c * C_E
            pltpu.async_copy(src_hbm.at[pl.ds(off, C_E)], src_idx.at[b],
                             sem_idx.at[b])
            pltpu.async_copy(dst_hbm.at[pl.ds(off, C_E)], dst_idx.at[b],
                             sem_idx.at[b])

        def wait_idx(c, b):
            off = base + c * C_E
            pltpu.make_async_copy(src_hbm.at[pl.ds(off, C_E)], src_idx.at[b],
                                  sem_idx.at[b]).wait()
            pltpu.make_async_copy(dst_hbm.at[pl.ds(off, C_E)], dst_idx.at[b],
                                  sem_idx.at[b]).wait()

        def out_rows(c):
            return eg * (E_GRP // L) + c * R

        def wait_out(c, b):
            pltpu.make_async_copy(
                acc_sh.at[pl.ds((slot + b) * R, R), :],
                out_hbm.at[pl.ds(out_rows(c), R), :],
                sem_out.at[b]).wait()

        start_idx(0, 0)
        pltpu.sync_copy(rows_hbm, rows_v)
        pltpu.sync_copy(hp_hbm.at[pl.ds(t * P, P), :], table_v)

        def chunk_body(c, carry):
            b = lax.rem(c, 2)
            wait_idx(c, b)

            @pl.when(c + 1 < N_CH)
            def _():
                start_idx(c + 1, 1 - b)

            @plsc.parallel_loop(0, G, unroll=4)
            def group_body(g):
                sv = src_idx[b, pl.ds(g * L, L)]
                dv = dst_idx[b, pl.ds(g * L, L)]
                prods = []
                for p in range(P):
                    pc = jnp.full((L,), p, jnp.int32)
                    ws = plsc.load_gather(table_v, [pc, sv])
                    wd = plsc.load_gather(table_v, [pc, dv])
                    sb = plsc.bitcast(ws, jnp.bfloat16)
                    db = plsc.bitcast(wd, jnp.bfloat16)
                    prods.append(sb * db)
                while len(prods) > 1:
                    prods = [x + y for x, y in zip(prods[::2], prods[1::2])]
                accw = plsc.bitcast(prods[0], jnp.int32)
                hi = plsc.bitcast(accw & _MASKHI, jnp.float32)
                lo = plsc.bitcast(accw << 16, jnp.float32)
                partial_v[b, g] = hi + lo

            @pl.when(jnp.logical_and(t == 0, c >= 2))
            def _():
                wait_out(c - 2, b)

            @pl.when(t == 0)
            def _():
                pltpu.sync_copy(partial_v.at[b],
                                acc_sh.at[pl.ds((slot + b) * R, R), :])

            plsc.subcore_barrier()

            @pl.when(t > 0)
            def _():
                pltpu.sync_copy(partial_v.at[b],
                                acc_sh.at[rows_v.at[slot + b]], add=True)

            plsc.subcore_barrier()

            @pl.when(t == 0)
            def _():
                pltpu.async_copy(acc_sh.at[pl.ds((slot + b) * R, R), :],
                                 out_hbm.at[pl.ds(out_rows(c), R), :],
                                 sem_out.at[b])
            return carry

        lax.fori_loop(0, N_CH, chunk_body, 0)

        @pl.when(t == 0)
        def _():
            for c in (N_CH - 2, N_CH - 1):
                wait_out(c, c % 2)

    return edge_score


_edge_score = _make_sc_kernel()

import numpy as _np

_ROWS = _np.arange(4 * R, dtype=_np.int32).reshape(4, R)


def kernel(h, edge_index):
    src = edge_index[0].astype(jnp.int32)
    dst = edge_index[1].astype(jnp.int32)
    scores = _edge_score(_pack(h), src, dst, _ROWS)
    return scores.reshape(E, 1)


# speedup vs baseline: 3.3529x; 3.3529x over previous
import functools

import jax
import jax.numpy as jnp
from jax import lax
from jax.experimental import pallas as pl
from jax.experimental.pallas import tpu as pltpu
from jax.experimental.pallas import tpu_sc as plsc

E = 320000
D = 128
N = 10000

_info = plsc.get_sparse_core_info()
NC, NS, L = _info.num_cores, _info.num_subcores, _info.num_lanes
NT = 8
NEG = 4
P = D // 2 // NT
E_GRP = E // NEG
C_E = 4000
N_CH = E_GRP // C_E
G = C_E // L
R = C_E // L
_MASKHI = -65536


def _pack_body(ht_ref, o_ref):
    x = ht_ref[...]
    u16 = lax.bitcast_convert_type(x.astype(jnp.bfloat16), jnp.uint16)
    u = u16.astype(jnp.int32).reshape(NT * P, 2, N)
    o_ref[...] = (u[:, 0, :] << 16) | u[:, 1, :]


_pack = pl.pallas_call(
    _pack_body,
    out_shape=jax.ShapeDtypeStruct((NT * P, N), jnp.int32),
)


def _make_sc_kernel():
    mesh = plsc.VectorSubcoreMesh(core_axis_name="c", subcore_axis_name="s")

    @functools.partial(
        pl.kernel,
        mesh=mesh,
        out_type=jax.ShapeDtypeStruct((E // L, L), jnp.float32),
        compiler_params=pltpu.CompilerParams(
            needs_layout_passes=False, use_tc_tiling_on_sc=False
        ),
        scratch_types=[
            pltpu.VMEM((P, N), jnp.int32),
            pltpu.VMEM((2, C_E), jnp.int32),
            pltpu.VMEM((2, C_E), jnp.int32),
            pltpu.VMEM((2, R, L), jnp.float32),
            pltpu.VMEM((4, R), jnp.int32),
            pltpu.VMEM_SHARED((4 * R, L), jnp.float32),
            pltpu.SemaphoreType.DMA((2,)),
            pltpu.SemaphoreType.DMA((2,)),
        ],
    )
    def edge_score(hp_hbm, src_hbm, dst_hbm, rows_hbm, out_hbm,
                   table_v, src_idx, dst_idx, partial_v, rows_v, acc_sh,
                   sem_idx, sem_out):
        core = lax.axis_index("c")
        s = lax.axis_index("s")
        t = lax.rem(s, NT)
        leg = s // NT
        eg = core * 2 + leg
        slot = leg * 2
        base = eg * E_GRP

        def start_idx(c, b):
            off = base + c * C_E
            pltpu.async_copy(src_hbm.at[pl.ds(off, C_E)], src_idx.at[b],
                             sem_idx.at[b])
            pltpu.async_copy(dst_hbm.at[pl.ds(off, C_E)], dst_idx.at[b],
                             sem_idx.at[b])

        def wait_idx(c, b):
            off = base + c * C_E
            pltpu.make_async_copy(src_hbm.at[pl.ds(off, C_E)], src_idx.at[b],
                                  sem_idx.at[b]).wait()
            pltpu.make_async_copy(dst_hbm.at[pl.ds(off, C_E)], dst_idx.at[b],
                                  sem_idx.at[b]).wait()

        def out_rows(c):
            return eg * (E_GRP // L) + c * R

        def wait_out(c, b):
            pltpu.make_async_copy(
                acc_sh.at[pl.ds((slot + b) * R, R), :],
                out_hbm.at[pl.ds(out_rows(c), R), :],
                sem_out.at[b]).wait()

        start_idx(0, 0)
        pltpu.sync_copy(rows_hbm, rows_v)
        pltpu.sync_copy(hp_hbm.at[pl.ds(t * P, P), :], table_v)

        def chunk_body(c, carry):
            b = lax.rem(c, 2)
            wait_idx(c, b)

            @pl.when(c + 1 < N_CH)
            def _():
                start_idx(c + 1, 1 - b)

            @plsc.parallel_loop(0, G, unroll=4)
            def group_body(g):
                sv = src_idx[b, pl.ds(g * L, L)]
                dv = dst_idx[b, pl.ds(g * L, L)]
                prods = []
                for p in range(P):
                    pc = jnp.full((L,), p, jnp.int32)
                    ws = plsc.load_gather(table_v, [pc, sv])
                    wd = plsc.load_gather(table_v, [pc, dv])
                    sb = plsc.bitcast(ws, jnp.bfloat16)
                    db = plsc.bitcast(wd, jnp.bfloat16)
                    prods.append(sb * db)
                acc = prods[0] + prods[1]
                for p in range(2, P):
                    acc = acc + prods[p]
                accw = plsc.bitcast(acc, jnp.int32)
                hi = plsc.bitcast(accw & _MASKHI, jnp.float32)
                lo = plsc.bitcast(accw << 16, jnp.float32)
                partial_v[b, g] = hi + lo

            @pl.when(jnp.logical_and(t == 0, c >= 2))
            def _():
                wait_out(c - 2, b)

            @pl.when(t == 0)
            def _():
                pltpu.sync_copy(partial_v.at[b],
                                acc_sh.at[pl.ds((slot + b) * R, R), :])

            plsc.subcore_barrier()

            @pl.when(t > 0)
            def _():
                pltpu.sync_copy(partial_v.at[b],
                                acc_sh.at[rows_v.at[slot + b]], add=True)

            plsc.subcore_barrier()

            @pl.when(t == 0)
            def _():
                pltpu.async_copy(acc_sh.at[pl.ds((slot + b) * R, R), :],
                                 out_hbm.at[pl.ds(out_rows(c), R), :],
                                 sem_out.at[b])
            return carry

        lax.fori_loop(0, N_CH, chunk_body, 0)

        @pl.when(t == 0)
        def _():
            for c in (N_CH - 2, N_CH - 1):
                wait_out(c, c % 2)

    return edge_score


_edge_score = _make_sc_kernel()

import numpy as _np

_ROWS = _np.arange(4 * R, dtype=_np.int32).reshape(4, R)


def kernel(h, edge_index):
    src = edge_index[0].astype(jnp.int32)
    dst = edge_index[1].astype(jnp.int32)
    scores = _edge_score(_pack(h.T), src, dst, _ROWS)
    return scores.reshape(E, 1)
